# baseline (speedup 1.0000x reference)
import jax
import jax.numpy as jnp
from jax import lax
from jax.experimental import pallas as pl
from jax.experimental.pallas import tpu as pltpu
from jax.experimental.pallas import tpu_sc as plsc

LANES = 16
LATENT = 32
NUM_WORKERS = 32
B_PER_W = 512


def _sc_body(uid_hbm, iid_hbm, eu, ei, out_hbm,
             uidx_v, iidx_v, u_rows, i_rows, out_v, sem):
    wid = lax.axis_index("s") * 2 + lax.axis_index("c")
    base = wid * B_PER_W

    pltpu.sync_copy(uid_hbm.at[pl.ds(base, B_PER_W)], uidx_v)
    pltpu.sync_copy(iid_hbm.at[pl.ds(base, B_PER_W)], iidx_v)

    pltpu.async_copy(eu.at[uidx_v], u_rows, sem)
    pltpu.async_copy(ei.at[iidx_v], i_rows, sem)
    pltpu.make_async_copy(eu.at[uidx_v], u_rows, sem).wait()
    pltpu.make_async_copy(ei.at[iidx_v], i_rows, sem).wait()

    lane_iota = lax.iota(jnp.int32, LANES)

    def chunk(ch, carry):
        rows = ch * LANES + lane_iota
        acc = jnp.zeros((LANES,), jnp.float32)
        for d in range(LATENT):
            cols = jnp.full((LANES,), d, jnp.int32)
            uv = plsc.load_gather(u_rows, [rows, cols])
            iv = plsc.load_gather(i_rows, [rows, cols])
            acc = acc + uv * iv
        out_v[pl.ds(ch * LANES, LANES)] = acc
        return carry

    lax.fori_loop(0, B_PER_W // LANES, chunk, 0)

    pltpu.sync_copy(out_v, out_hbm.at[pl.ds(base, B_PER_W)])


def kernel(user_id, item_id, emb_user, emb_item):
    batch = user_id.shape[0]
    uid = user_id.astype(jnp.int32)
    iid = item_id.astype(jnp.int32)

    mesh = plsc.VectorSubcoreMesh(core_axis_name="c", subcore_axis_name="s")
    run = pl.kernel(
        _sc_body,
        out_type=jax.ShapeDtypeStruct((batch,), jnp.float32),
        mesh=mesh,
        compiler_params=pltpu.CompilerParams(
            use_tc_tiling_on_sc=False, needs_layout_passes=False),
        scratch_types=[
            pltpu.VMEM((B_PER_W,), jnp.int32),
            pltpu.VMEM((B_PER_W,), jnp.int32),
            pltpu.VMEM((B_PER_W, LATENT), jnp.float32),
            pltpu.VMEM((B_PER_W, LATENT), jnp.float32),
            pltpu.VMEM((B_PER_W,), jnp.float32),
            pltpu.SemaphoreType.DMA,
        ],
    )
    out = run(uid, iid, emb_user, emb_item)
    return out.reshape(batch, 1)
